# Initial kernel scaffold; baseline (speedup 1.0000x reference)
import functools

import jax
import jax.numpy as jnp
from jax import lax
from jax.experimental import pallas as pl
from jax.experimental.pallas import tpu as pltpu
from jax.experimental.pallas import tpu_sc as plsc

LEVELS = 16
BASE_RESO = 16
TABLE_SIZE = 524288
MASK = 524287
PRIME32 = -1640531535
B = 4
N = 65536
BN = B * N
NW = 32
PER_W = BN // NW
C = 2048
NSUB = PER_W // C
JROWS = C // 128


def _body(px_hbm, table_hbm, out_hbm, px_v, idx_v, gbuf, sem):
    nc = 2
    wid = lax.axis_index("s") * nc + lax.axis_index("c")
    wbase = wid * PER_W
    rowoff = (wbase // N) * TABLE_SIZE

    scales = [jnp.float32(BASE_RESO * (1 << l) - 1) for l in range(LEVELS)]

    def sub_chunk(s, _):
        base = wbase + s * C
        pltpu.sync_copy(px_hbm.at[pl.ds(base, C)], px_v)

        def idx_row(j, _):
            for i2 in range(8):
                x = px_v[pl.ds(j * 128 + i2 * 16, 16)]
                for l in range(LEVELS):
                    c0 = (x * scales[l]).astype(jnp.int32)
                    h = (c0 ^ (c0 * jnp.int32(PRIME32))) & jnp.int32(MASK)
                    idx_v[l, j, pl.ds(i2 * 16, 16)] = h + rowoff
            for l in range(LEVELS):
                pltpu.make_async_copy(
                    table_hbm.at[idx_v.at[l, j]],
                    gbuf.at[l, pl.ds(j * 128, 128)],
                    sem,
                ).start()
            return 0

        lax.fori_loop(0, JROWS, idx_row, 0)

        def drain_row(j, _):
            for l in range(LEVELS):
                pltpu.make_async_copy(
                    table_hbm.at[idx_v.at[l, j]],
                    gbuf.at[l, pl.ds(j * 128, 128)],
                    sem,
                ).wait()
            return 0

        lax.fori_loop(0, JROWS, drain_row, 0)

        for l in range(LEVELS):
            pltpu.sync_copy(gbuf.at[l], out_hbm.at[l, pl.ds(base, C)])
        return 0

    lax.fori_loop(0, NSUB, sub_chunk, 0)


@jax.jit
def kernel(p, enc):
    px = p[..., 0].reshape(BN)
    table = enc.reshape(B * TABLE_SIZE, 2)
    mesh = plsc.VectorSubcoreMesh(core_axis_name="c", subcore_axis_name="s")
    run = functools.partial(
        pl.kernel,
        mesh=mesh,
        out_type=jax.ShapeDtypeStruct((LEVELS, BN, 2), jnp.float32),
        scratch_types=[
            pltpu.VMEM((C,), jnp.float32),
            pltpu.VMEM((LEVELS, JROWS, 128), jnp.int32),
            pltpu.VMEM((LEVELS, C, 2), jnp.float32),
            pltpu.SemaphoreType.DMA,
        ],
    )(_body)
    out = run(px, table)
    return out.transpose(1, 0, 2).reshape(B, N, LEVELS * 2)


# speedup vs baseline: 4.7234x; 4.7234x over previous
import functools

import jax
import jax.numpy as jnp
from jax import lax
from jax.experimental import pallas as pl
from jax.experimental.pallas import tpu as pltpu
from jax.experimental.pallas import tpu_sc as plsc

LEVELS = 16
BASE_RESO = 16
TABLE_SIZE = 524288
MASK = 524287
PRIME32 = -1640531535
B = 4
N = 65536
BN = B * N
NW = 32
PER_W = BN // NW
C = 2048
NSUB = PER_W // C
JROWS = C // 128


def _body(px_hbm, table_hbm, out_hbm, px_v, idx_v, gb0, gb1, sem):
    wid = lax.axis_index("s") * jnp.int32(2) + lax.axis_index("c")
    wbase = wid * jnp.int32(PER_W)
    eoff = (wbase // jnp.int32(N)) * jnp.int32(TABLE_SIZE * 2)

    scales = [jnp.float32(BASE_RESO * (1 << l) - 1) for l in range(LEVELS)]
    gbufs = (gb0, gb1)

    def sub_chunk(s, _):
        base = wbase + s * jnp.int32(C)
        pltpu.sync_copy(px_hbm.at[pl.ds(base, C)], px_v)

        def idx_row(j, _):
            for i2 in range(8):
                x = px_v[pl.ds(j * jnp.int32(128) + jnp.int32(i2 * 16), 16)]
                for l in range(LEVELS):
                    c0 = (x * scales[l]).astype(jnp.int32)
                    h = (c0 ^ (c0 * jnp.int32(PRIME32))) & jnp.int32(MASK)
                    e0 = (h << jnp.int32(1)) + eoff
                    idx_v[l, 0, j, pl.ds(i2 * 16, 16)] = e0
                    idx_v[l, 1, j, pl.ds(i2 * 16, 16)] = e0 + jnp.int32(1)
            return 0

        lax.fori_loop(0, JROWS, idx_row, 0)

        def fire(l):
            def go(j, _):
                for v in range(2):
                    pltpu.make_async_copy(
                        table_hbm.at[idx_v.at[l, v, j]],
                        gbufs[l % 2].at[v, pl.ds(j * jnp.int32(128), 128)],
                        sem,
                    ).start()
                return 0
            lax.fori_loop(0, JROWS, go, 0)

        def drain_and_flush(l):
            def dr(j, _):
                for v in range(2):
                    pltpu.make_async_copy(
                        table_hbm.at[idx_v.at[l, v, j]],
                        gbufs[l % 2].at[v, pl.ds(j * jnp.int32(128), 128)],
                        sem,
                    ).wait()
                return 0
            lax.fori_loop(0, JROWS, dr, 0)
            pltpu.sync_copy(gbufs[l % 2], out_hbm.at[l, :, pl.ds(base, C)])

        fire(0)
        for l in range(1, LEVELS):
            fire(l)
            drain_and_flush(l - 1)
        drain_and_flush(LEVELS - 1)
        return 0

    lax.fori_loop(0, NSUB, sub_chunk, 0)


@jax.jit
def _run(p, enc):
    px = p[..., 0].reshape(BN)
    table = enc.reshape(B * TABLE_SIZE * 2)
    mesh = plsc.VectorSubcoreMesh(core_axis_name="c", subcore_axis_name="s")
    run = functools.partial(
        pl.kernel,
        mesh=mesh,
        out_type=jax.ShapeDtypeStruct((LEVELS, 2, BN), jnp.float32),
        scratch_types=[
            pltpu.VMEM((C,), jnp.float32),
            pltpu.VMEM((LEVELS, 2, JROWS, 128), jnp.int32),
            pltpu.VMEM((2, C), jnp.float32),
            pltpu.VMEM((2, C), jnp.float32),
            pltpu.SemaphoreType.DMA,
        ],
        compiler_params=pltpu.CompilerParams(use_tc_tiling_on_sc=False),
    )(_body)
    out = run(px, table)
    return out.transpose(2, 0, 1).reshape(B, N, LEVELS * 2)


def kernel(p, enc):
    with jax.enable_x64(False):
        return _run(p, enc)


# speedup vs baseline: 6.6290x; 1.4034x over previous
import functools

import jax
import jax.numpy as jnp
from jax import lax
from jax.experimental import pallas as pl
from jax.experimental.pallas import tpu as pltpu
from jax.experimental.pallas import tpu_sc as plsc

LEVELS = 16
BASE_RESO = 16
TABLE_SIZE = 524288
ROWS8 = TABLE_SIZE // 4
MASK = 524287
PRIME32 = -1640531535
B = 4
N = 65536
BN = B * N
NW = 32
PER_W = BN // NW
C = 1024
NSUB = PER_W // C
JROWS = C // 128
G16 = C // 16


def _body(px_hbm, table_hbm, out_hbm, pxbuf, idx_v, sub_v, gb, obuf,
          gsem, psem):
    wid = lax.axis_index("s") * jnp.int32(2) + lax.axis_index("c")
    wbase = wid * jnp.int32(PER_W)
    roff8 = (wbase // jnp.int32(N)) * jnp.int32(ROWS8)

    scales = [jnp.float32(BASE_RESO * (1 << l) - 1) for l in range(LEVELS)]
    iota = lax.iota(jnp.int32, 16)

    def px_copy(s):
        return pltpu.make_async_copy(
            px_hbm.at[pl.ds(wbase + s * jnp.int32(C), C)],
            pxbuf.at[s & jnp.int32(1)],
            psem,
        )

    px_copy(jnp.int32(0)).start()

    def sub_chunk(s, _):
        base = wbase + s * jnp.int32(C)
        sel = s & jnp.int32(1)
        px_copy(s).wait()

        @pl.when(s < jnp.int32(NSUB - 1))
        def _():
            px_copy(s + jnp.int32(1)).start()

        def idx_g(g, _):
            j = g >> jnp.int32(3)
            k = (g & jnp.int32(7)) * jnp.int32(16)
            x = pxbuf[sel, pl.ds(g * jnp.int32(16), 16)]
            for l in range(LEVELS):
                c0 = (x * scales[l]).astype(jnp.int32)
                h = (c0 ^ (c0 * jnp.int32(PRIME32))) & jnp.int32(MASK)
                idx_v[l, j, pl.ds(k, 16)] = (h >> jnp.int32(2)) + roff8
                sub_v[l, pl.ds(g * jnp.int32(16), 16)] = (
                    (h & jnp.int32(3)) << jnp.int32(1))
            return 0

        lax.fori_loop(0, G16, idx_g, 0)

        def fire(l):
            def go(j, _):
                pltpu.make_async_copy(
                    table_hbm.at[idx_v.at[l, j]],
                    gb.at[l % 2, pl.ds(j * jnp.int32(128), 128)],
                    gsem,
                ).start()
                return 0
            lax.fori_loop(0, JROWS, go, 0)

        def drain(l):
            def dr(j, _):
                pltpu.make_async_copy(
                    table_hbm.at[idx_v.at[l, j]],
                    gb.at[l % 2, pl.ds(j * jnp.int32(128), 128)],
                    gsem,
                ).wait()
                return 0
            lax.fori_loop(0, JROWS, dr, 0)

        def extract(l):
            lsel = iota * jnp.int32(0) + jnp.int32(l % 2)

            def ex(g, _):
                gbase = g * jnp.int32(16)
                sub = sub_v[l, pl.ds(gbase, 16)]
                rows = gbase + iota
                v0 = plsc.load_gather(gb, [lsel, rows, sub])
                v1 = plsc.load_gather(gb, [lsel, rows, sub + jnp.int32(1)])
                obuf[l, 0, pl.ds(gbase, 16)] = v0
                obuf[l, 1, pl.ds(gbase, 16)] = v1
                return 0

            lax.fori_loop(0, G16, ex, 0)

        fire(0)
        for l in range(1, LEVELS):
            fire(l)
            drain(l - 1)
            extract(l - 1)
        drain(LEVELS - 1)
        extract(LEVELS - 1)

        pltpu.sync_copy(obuf, out_hbm.at[:, :, pl.ds(base, C)])
        return 0

    lax.fori_loop(0, NSUB, sub_chunk, 0)


@jax.jit
def _run(p, enc):
    px = p[..., 0].reshape(BN)
    table = enc.reshape(B * ROWS8, 8)
    mesh = plsc.VectorSubcoreMesh(core_axis_name="c", subcore_axis_name="s")
    run = functools.partial(
        pl.kernel,
        mesh=mesh,
        out_type=jax.ShapeDtypeStruct((LEVELS, 2, BN), jnp.float32),
        scratch_types=[
            pltpu.VMEM((2, C), jnp.float32),
            pltpu.VMEM((LEVELS, JROWS, 128), jnp.int32),
            pltpu.VMEM((LEVELS, C), jnp.int32),
            pltpu.VMEM((2, C, 8), jnp.float32),
            pltpu.VMEM((LEVELS, 2, C), jnp.float32),
            pltpu.SemaphoreType.DMA,
            pltpu.SemaphoreType.DMA,
        ],
        compiler_params=pltpu.CompilerParams(
            use_tc_tiling_on_sc=False, needs_layout_passes=False),
    )(_body)
    out = run(px, table)
    return out.transpose(2, 0, 1).reshape(B, N, LEVELS * 2)


def kernel(p, enc):
    with jax.enable_x64(False):
        return _run(p, enc)
